# baseline (speedup 1.0000x reference)
import functools

import jax
import jax.numpy as jnp
from jax import lax
from jax.experimental import pallas as pl
from jax.experimental.pallas import tpu as pltpu
from jax.experimental.pallas import tpu_sc as plsc

D = 1024
EPS = 1e-6
L = 16
NB = D // L
NW = 32
ROWS_PER_W = 512
C = 32
G = ROWS_PER_W // C


def _lane_sum(x):
    lanes = lax.iota(jnp.int32, L)
    dn = lax.GatherDimensionNumbers(
        offset_dims=(), collapsed_slice_dims=(0,), start_index_map=(0,)
    )
    for sh in (8, 4, 2, 1):
        perm = lax.bitwise_xor(lanes, jnp.int32(sh))
        x = x + lax.gather(
            x,
            perm[:, None],
            dn,
            slice_sizes=(1,),
            mode=lax.GatherScatterMode.PROMISE_IN_BOUNDS,
        )
    return x


def _rsqrt_vec(x):
    i = lax.bitcast_convert_type(x, jnp.int32)
    i = jnp.int32(0x5F3759DF) - lax.shift_right_logical(i, 1)
    y = lax.bitcast_convert_type(i, jnp.float32)
    for _ in range(3):
        y = y * (1.5 - 0.5 * x * y * y)
    return y


def _body(table_h, idx_h, g_h, b_h, out_h, idx_v, rows_v, g_v, b_v, gsem):
    cid = lax.axis_index("c")
    sid = lax.axis_index("s")
    wid = sid * 2 + cid
    base = wid * ROWS_PER_W

    pltpu.sync_copy(idx_h.at[pl.ds(base, ROWS_PER_W)], idx_v)
    pltpu.sync_copy(g_h, g_v)
    pltpu.sync_copy(b_h, b_v)

    def chunk_fn(g, carry):
        row0 = pl.multiple_of(g * C, C)
        pltpu.async_copy(
            table_h.at[idx_v.at[pl.ds(row0, C)]], rows_v, gsem
        ).wait()

        def row_fn(r, c2):
            acc = [jnp.zeros((L,), jnp.float32) for _ in range(4)]
            accsq = [jnp.zeros((L,), jnp.float32) for _ in range(4)]
            for j in range(NB):
                v = rows_v[r, pl.ds(j * L, L)]
                acc[j % 4] = acc[j % 4] + v
                accsq[j % 4] = accsq[j % 4] + v * v
            s = (acc[0] + acc[1]) + (acc[2] + acc[3])
            sq = (accsq[0] + accsq[1]) + (accsq[2] + accsq[3])
            mean_vec = _lane_sum(s) * (1.0 / D)
            var_vec = _lane_sum(sq) * (1.0 / D) - mean_vec * mean_vec
            rstd_vec = _rsqrt_vec(var_vec + EPS)
            for j in range(NB):
                v = rows_v[r, pl.ds(j * L, L)]
                gv = g_v[pl.ds(j * L, L)]
                bv = b_v[pl.ds(j * L, L)]
                rows_v[r, pl.ds(j * L, L)] = (v - mean_vec) * rstd_vec * gv + bv
            return c2

        lax.fori_loop(0, C, row_fn, 0)
        pltpu.sync_copy(rows_v, out_h.at[pl.ds(base + row0, C)])
        return carry

    lax.fori_loop(0, G, chunk_fn, 0)


@jax.jit
def _emb_ln(table, idx, gamma, beta):
    mesh = plsc.VectorSubcoreMesh(core_axis_name="c", subcore_axis_name="s")
    return pl.kernel(
        _body,
        out_type=jax.ShapeDtypeStruct((idx.shape[0], D), jnp.float32),
        mesh=mesh,
        scratch_types=[
            pltpu.VMEM((ROWS_PER_W,), jnp.int32),
            pltpu.VMEM((C, D), jnp.float32),
            pltpu.VMEM((D,), jnp.float32),
            pltpu.VMEM((D,), jnp.float32),
            pltpu.SemaphoreType.DMA,
        ],
    )(table, idx, gamma, beta)


def kernel(src, table, gamma, beta):
    idx = src.reshape(-1).astype(jnp.int32)
    out = _emb_ln(table, idx, gamma, beta)
    return out.reshape(src.shape + (D,))
